# Initial kernel scaffold; baseline (speedup 1.0000x reference)
import jax
import jax.numpy as jnp
from jax.experimental import pallas as pl


def kernel(input_ids, table):
    raise NotImplementedError("write your pallas kernel here")


# speedup vs baseline: 1.2962x; 1.2962x over previous
import functools

import jax
import jax.numpy as jnp
from jax import lax
from jax.experimental import pallas as pl
from jax.experimental.pallas import tpu as pltpu
from jax.experimental.pallas import tpu_sc as plsc

BATCH = 1024
SEQ = 77
VOCAB = 49408
DIM = 768

NC = 2
NS = 16
NW = NC * NS

B = BATCH * SEQ
B_PER_W = B // NW
CHUNK = 56
NCHUNK = B_PER_W // CHUNK
NBUF = 2
NOUTER = NCHUNK // NBUF - 1

_mesh = plsc.VectorSubcoreMesh(
    core_axis_name="c", subcore_axis_name="s", num_cores=NC, num_subcores=NS
)


@functools.partial(
    pl.kernel,
    mesh=_mesh,
    out_type=jax.ShapeDtypeStruct((B, DIM), jnp.float32),
    scratch_types=[
        pltpu.VMEM((B_PER_W,), jnp.int32),
        pltpu.VMEM((NBUF, CHUNK, DIM), jnp.float32),
        pltpu.SemaphoreType.DMA,
        pltpu.SemaphoreType.DMA,
    ],
)
def _sc_gather(idx_hbm, table_hbm, out_hbm, idx_v, bufs, gsem, ssem):
    wid = lax.axis_index("s") * NC + lax.axis_index("c")
    base = wid * B_PER_W

    pltpu.sync_copy(idx_hbm.at[pl.ds(base, B_PER_W)], idx_v)

    def gather_start(c, b):
        pltpu.async_copy(
            table_hbm.at[idx_v.at[pl.ds(c * CHUNK, CHUNK)]], bufs.at[b], gsem
        )

    def gather_wait(b):
        pltpu.make_async_copy(
            table_hbm.at[idx_v.at[pl.ds(0, CHUNK)]], bufs.at[b], gsem
        ).wait()

    def store_start(c, b):
        pltpu.async_copy(
            bufs.at[b], out_hbm.at[pl.ds(base + c * CHUNK, CHUNK)], ssem
        )

    def store_wait(b):
        pltpu.make_async_copy(
            bufs.at[b], out_hbm.at[pl.ds(base, CHUNK)], ssem
        ).wait()

    for b in range(NBUF):
        gather_start(b, b)

    def body(g, _):
        for b in range(NBUF):
            c = NBUF * g + b
            gather_wait(b)
            store_start(c, b)
        for b in range(NBUF):
            store_wait(b)
            gather_start(NBUF * (g + 1) + b, b)
        return ()

    lax.fori_loop(0, NOUTER, body, (), unroll=False)

    for b in range(NBUF):
        c = NBUF * NOUTER + b
        gather_wait(b)
        store_start(c, b)
    for b in range(NBUF):
        store_wait(b)


def kernel(input_ids, table):
    idx = input_ids.reshape(-1)
    out = _sc_gather(idx, table)
    return out.reshape(BATCH, SEQ, DIM)


# speedup vs baseline: 1.2992x; 1.0023x over previous
import functools

import jax
import jax.numpy as jnp
from jax import lax
from jax.experimental import pallas as pl
from jax.experimental.pallas import tpu as pltpu
from jax.experimental.pallas import tpu_sc as plsc

BATCH = 1024
SEQ = 77
VOCAB = 49408
DIM = 768

NC = 2
NS = 16
NW = NC * NS

B = BATCH * SEQ
B_PER_W = B // NW
CHUNK = 16
NCHUNK = B_PER_W // CHUNK
NBUF = 7
NOUTER = NCHUNK // NBUF - 1

_mesh = plsc.VectorSubcoreMesh(
    core_axis_name="c", subcore_axis_name="s", num_cores=NC, num_subcores=NS
)


@functools.partial(
    pl.kernel,
    mesh=_mesh,
    out_type=jax.ShapeDtypeStruct((B, DIM), jnp.float32),
    scratch_types=[
        pltpu.VMEM((B_PER_W,), jnp.int32),
        pltpu.VMEM((NBUF, CHUNK, DIM), jnp.float32),
        pltpu.SemaphoreType.DMA,
        pltpu.SemaphoreType.DMA,
    ],
)
def _sc_gather(idx_hbm, table_hbm, out_hbm, idx_v, bufs, gsem, ssem):
    wid = lax.axis_index("s") * NC + lax.axis_index("c")
    base = wid * B_PER_W

    pltpu.sync_copy(idx_hbm.at[pl.ds(base, B_PER_W)], idx_v)

    def gather_start(c, b):
        pltpu.async_copy(
            table_hbm.at[idx_v.at[pl.ds(c * CHUNK, CHUNK)]], bufs.at[b], gsem
        )

    def gather_wait(b):
        pltpu.make_async_copy(
            table_hbm.at[idx_v.at[pl.ds(0, CHUNK)]], bufs.at[b], gsem
        ).wait()

    def store_start(c, b):
        pltpu.async_copy(
            bufs.at[b], out_hbm.at[pl.ds(base + c * CHUNK, CHUNK)], ssem
        )

    def store_wait(b):
        pltpu.make_async_copy(
            bufs.at[b], out_hbm.at[pl.ds(base, CHUNK)], ssem
        ).wait()

    for b in range(NBUF):
        gather_start(b, b)

    def body(g, _):
        for b in range(NBUF):
            c = NBUF * g + b
            gather_wait(b)
            store_start(c, b)
        for b in range(NBUF):
            store_wait(b)
            gather_start(NBUF * (g + 1) + b, b)
        return ()

    lax.fori_loop(0, NOUTER, body, (), unroll=False)

    for b in range(NBUF):
        c = NBUF * NOUTER + b
        gather_wait(b)
        store_start(c, b)
    for b in range(NBUF):
        store_wait(b)


def kernel(input_ids, table):
    idx = input_ids.reshape(-1)
    out = _sc_gather(idx, table)
    return out.reshape(BATCH, SEQ, DIM)


# speedup vs baseline: 3.7327x; 2.8731x over previous
import functools

import jax
import jax.numpy as jnp
from jax import lax
from jax.experimental import pallas as pl
from jax.experimental.pallas import tpu as pltpu
from jax.experimental.pallas import tpu_sc as plsc

BATCH = 1024
SEQ = 77
VOCAB = 49408
DIM = 768

NC = 2
NS = 16
NW = NC * NS

B = BATCH * SEQ
B_PER_W = B // NW
CHUNK = 16
NCHUNK = B_PER_W // CHUNK
NBUF = 7
NOUTER = NCHUNK // NBUF - 1

_mesh = plsc.VectorSubcoreMesh(
    core_axis_name="c", subcore_axis_name="s", num_cores=NC, num_subcores=NS
)


@functools.partial(
    pl.kernel,
    mesh=_mesh,
    out_type=jax.ShapeDtypeStruct((B, DIM), jnp.float32),
    scratch_types=[
        pltpu.VMEM((B_PER_W,), jnp.int32),
        pltpu.VMEM((NBUF, CHUNK, DIM), jnp.float32),
        pltpu.SemaphoreType.DMA,
        pltpu.SemaphoreType.DMA,
    ],
)
def _sc_gather(idx_hbm, table_hbm, out_hbm, idx_v, bufs, gsem, ssem):
    wid = lax.axis_index("s") * NC + lax.axis_index("c")
    base = wid * B_PER_W

    pltpu.sync_copy(idx_hbm.at[pl.ds(base, B_PER_W)], idx_v)

    def gather_start(c, b):
        pltpu.async_copy(
            table_hbm.at[idx_v.at[pl.ds(c * CHUNK, CHUNK)]], bufs.at[b], gsem
        )

    def gather_wait(b):
        pltpu.make_async_copy(
            table_hbm.at[idx_v.at[pl.ds(0, CHUNK)]], bufs.at[b], gsem
        ).wait()

    def store_start(c, b):
        pltpu.async_copy(
            bufs.at[b], out_hbm.at[pl.ds(base + c * CHUNK, CHUNK)], ssem
        )

    def store_wait(b):
        pltpu.make_async_copy(
            bufs.at[b], out_hbm.at[pl.ds(base, CHUNK)], ssem
        ).wait()

    for b in range(NBUF):
        gather_start(b, b)

    def body(g, _):
        for b in range(NBUF):
            c = NBUF * g + b
            gather_wait(b)
            store_start(c, b)
        for b in range(NBUF):
            store_wait(b)
            gather_start(NBUF * (g + 1) + b, b)
        return ()

    lax.fori_loop(0, NOUTER, body, (), unroll=False)

    for b in range(NBUF):
        c = NBUF * NOUTER + b
        gather_wait(b)
        store_start(c, b)
    for b in range(NBUF):
        store_wait(b)


def kernel(input_ids, table):
    idx = jnp.transpose(input_ids).reshape(-1)
    out = _sc_gather(idx, table)
    return out.reshape(SEQ, BATCH, DIM).transpose(1, 0, 2)
